# baseline (speedup 1.0000x reference)
import jax
import jax.numpy as jnp
from jax.experimental import pallas as pl
from jax.experimental.pallas import tpu as pltpu
from jax.experimental.pallas import tpu_sc as plsc

_NUM_HEADS = 16
_H = 32
_W = 32
_D = 2 * _W - 1
_NTOK = _H * _W
_WIDE_PAD = 3976


def _sc_body(wide_hbm, out_hbm, wide_v, strip_v, sem):
    n = jax.lax.axis_index("s")
    half = jax.lax.axis_index("c")
    row0 = half * 16

    pltpu.sync_copy(wide_hbm.at[n], wide_v)

    for iwl in range(16):
        base = 31 - (row0 + iwl)

        def body(dd, _, iwl=iwl, base=base):
            strip_v[iwl, pl.ds(dd * 32, 16)] = wide_v[pl.ds(dd * 63 + base, 16)]
            strip_v[iwl, pl.ds(dd * 32 + 16, 16)] = wide_v[
                pl.ds(dd * 63 + base + 16, 16)
            ]
            return 0

        jax.lax.fori_loop(0, _D, body, 0)

    copies = []
    for ih in range(_H):
        off = (31 - ih) * 32
        copies.append(
            pltpu.async_copy(
                strip_v.at[:, pl.ds(off, 1024)],
                out_hbm.at[0, n, pl.ds(ih * 32 + row0, 16), :],
                sem,
            )
        )
    for cp in copies:
        cp.wait()


def kernel(relative_position_bias_table, h, w):
    del h, w
    wide = jnp.pad(
        jnp.flip(relative_position_bias_table, 0).T,
        ((0, 0), (0, _WIDE_PAD - _D * _D)),
    )

    mesh = plsc.VectorSubcoreMesh(
        core_axis_name="c", subcore_axis_name="s", num_cores=2, num_subcores=16
    )
    out = pl.kernel(
        _sc_body,
        out_type=jax.ShapeDtypeStruct(
            (1, _NUM_HEADS, _NTOK, _NTOK), jnp.float32
        ),
        mesh=mesh,
        scratch_types=[
            pltpu.VMEM((_WIDE_PAD,), jnp.float32),
            pltpu.VMEM((16, _D * 32), jnp.float32),
            pltpu.SemaphoreType.DMA,
        ],
        compiler_params=pltpu.CompilerParams(use_tc_tiling_on_sc=False),
    )(wide)
    return out


# speedup vs baseline: 2.6909x; 2.6909x over previous
import jax
import jax.numpy as jnp
from jax.experimental import pallas as pl
from jax.experimental.pallas import tpu as pltpu

_NUM_HEADS = 16
_H = 32
_W = 32
_D = 2 * _W - 1


def _band_kernel(wide_ref, out_ref, strip_ref, sems):
    h = wide_ref[...][:, :, None, :]
    for k in range(5):
        s = 1 << k
        shifted = jnp.concatenate(
            [h[..., s:], jnp.zeros(h.shape[:-1] + (s,), h.dtype)], axis=-1
        )
        h = jnp.concatenate([shifted, h], axis=2)
    copies = []
    for k in range(4):
        for dp in range(k, _D):
            strip_ref[k, :, :, (dp - k) * 32:(dp - k + 1) * 32] = h[:, dp, :, :32]
        for ih in range(_H):
            r = 31 - ih
            if r % 4 != k:
                continue
            q = r // 4
            c = pltpu.make_async_copy(
                strip_ref.at[k, :, :, q * 128:q * 128 + 1024],
                out_ref.at[0, :, ih * 32:(ih + 1) * 32, :],
                sems.at[ih],
            )
            c.start()
            copies.append(c)
    for c in copies:
        c.wait()


def kernel(relative_position_bias_table, h, w):
    del h, w
    n_tok = _H * _W
    wide = jnp.flip(relative_position_bias_table, 0).T.reshape(
        _NUM_HEADS, _D, _D
    )
    wide = jnp.pad(wide, ((0, 0), (0, 0), (0, 1)))

    out = pl.pallas_call(
        _band_kernel,
        grid=(1,),
        in_specs=[pl.BlockSpec((_NUM_HEADS, _D, 64), lambda i: (0, 0, 0))],
        out_specs=pl.BlockSpec(memory_space=pl.MemorySpace.ANY),
        out_shape=jax.ShapeDtypeStruct(
            (1, _NUM_HEADS, n_tok, n_tok), jnp.float32
        ),
        scratch_shapes=[
            pltpu.VMEM((4, _NUM_HEADS, _W, 2048), jnp.float32),
            pltpu.SemaphoreType.DMA((_H,)),
        ],
    )(wide)
    return out


# speedup vs baseline: 2.9114x; 1.0820x over previous
import jax
import jax.numpy as jnp
from jax.experimental import pallas as pl
from jax.experimental.pallas import tpu as pltpu

_NUM_HEADS = 16
_H = 32
_W = 32
_D = 2 * _W - 1


def _band_kernel(wide_ref, out_ref, strip_ref, sems):
    h = wide_ref[...][:, None, :, :]
    for k in range(5):
        s = 1 << k
        shifted = jnp.concatenate(
            [h[..., s:], jnp.zeros(h.shape[:-1] + (s,), h.dtype)], axis=-1
        )
        h = jnp.concatenate([shifted, h], axis=1)
    copies = []
    for k in range(4):
        for dp in range(k, _D):
            p, half = divmod(dp, 2)
            strip_ref[k, :, :, (dp - k) * 32:(dp - k + 1) * 32] = h[
                :, :, p, half * 64:half * 64 + 32
            ]
        for ih in range(_H):
            r = 31 - ih
            if r % 4 != k:
                continue
            q = r // 4
            c = pltpu.make_async_copy(
                strip_ref.at[k, :, :, q * 128:q * 128 + 1024],
                out_ref.at[0, :, ih * 32:(ih + 1) * 32, :],
                sems.at[ih],
            )
            c.start()
            copies.append(c)
    for c in copies:
        c.wait()


def kernel(relative_position_bias_table, h, w):
    del h, w
    n_tok = _H * _W
    wide = jnp.flip(relative_position_bias_table, 0).T.reshape(
        _NUM_HEADS, _D, _D
    )
    wide = jnp.pad(wide, ((0, 0), (0, 1), (0, 1))).reshape(_NUM_HEADS, 32, 128)

    out = pl.pallas_call(
        _band_kernel,
        grid=(1,),
        in_specs=[pl.BlockSpec((_NUM_HEADS, 32, 128), lambda i: (0, 0, 0))],
        out_specs=pl.BlockSpec(memory_space=pl.MemorySpace.ANY),
        out_shape=jax.ShapeDtypeStruct(
            (1, _NUM_HEADS, n_tok, n_tok), jnp.float32
        ),
        scratch_shapes=[
            pltpu.VMEM((4, _NUM_HEADS, _W, 2048), jnp.float32),
            pltpu.SemaphoreType.DMA((_H,)),
        ],
    )(wide)
    return out
